# baseline (speedup 1.0000x reference)
import numpy as np
import jax
import jax.numpy as jnp
from jax.experimental import pallas as pl
from jax.experimental.pallas import tpu as pltpu

LEN = 128
NSEM = 8


def _frame_indices(T: int) -> np.ndarray:
    f = np.linspace(0.0, T - 1, LEN, dtype=np.float32)
    return np.clip(np.rint(f), 0, T - 1).astype(np.int32)


def kernel(x):
    T, C, H, W = x.shape
    idx = [int(v) for v in _frame_indices(T)]

    def body(x_ref, o_ref, sems):
        copies = [
            pltpu.make_async_copy(x_ref.at[idx[i]], o_ref.at[i], sems.at[i % NSEM])
            for i in range(LEN)
        ]
        for c in copies:
            c.start()
        for c in copies:
            c.wait()

    return pl.pallas_call(
        body,
        in_specs=[pl.BlockSpec(memory_space=pltpu.MemorySpace.HBM)],
        out_specs=pl.BlockSpec(memory_space=pltpu.MemorySpace.HBM),
        out_shape=jax.ShapeDtypeStruct((LEN, C, H, W), x.dtype),
        scratch_shapes=[pltpu.SemaphoreType.DMA((NSEM,))],
    )(x)


# speedup vs baseline: 17.3654x; 17.3654x over previous
import functools

import jax
import jax.numpy as jnp
from jax import lax
from jax.experimental import pallas as pl
from jax.experimental.pallas import tpu as pltpu
from jax.experimental.pallas import tpu_sc as plsc

LEN = 128
NW = 32


def kernel(x):
    T, C, H, W = x.shape
    chunks = LEN * C
    per_w = chunks // NW
    a, b = 2 * (T - 1), 2 * (LEN - 1)

    mesh = plsc.VectorSubcoreMesh(core_axis_name="c", subcore_axis_name="s")

    @functools.partial(
        pl.kernel,
        out_type=jax.ShapeDtypeStruct((LEN, C, H, W), x.dtype),
        mesh=mesh,
        scratch_types=[
            pltpu.VMEM((2, H, W), x.dtype),
            pltpu.SemaphoreType.DMA,
            pltpu.SemaphoreType.DMA,
            pltpu.SemaphoreType.DMA,
            pltpu.SemaphoreType.DMA,
        ],
    )
    def k(x_hbm, out_hbm, buf, si0, si1, so0, so1):
        wid = lax.axis_index("s") * 2 + lax.axis_index("c")
        base = wid * per_w
        sin = (si0, si1)
        sout = (so0, so1)

        def start_in(q, slot):
            o = base + q
            frame = o // C
            ch = o % C
            src = (frame * a + (LEN - 1)) // b
            return pltpu.async_copy(x_hbm.at[src, ch], buf.at[slot], sin[slot])

        def start_out(q, slot):
            o = base + q
            return pltpu.async_copy(buf.at[slot], out_hbm.at[o // C, o % C], sout[slot])

        in_cp = [None, None]
        out_cp = [None, None]
        in_cp[0] = start_in(0, 0)
        for q in range(per_w):
            slot = q % 2
            nxt = (q + 1) % 2
            if q + 1 < per_w:
                if q >= 1:
                    out_cp[nxt].wait()
                in_cp[nxt] = start_in(q + 1, nxt)
            in_cp[slot].wait()
            out_cp[slot] = start_out(q, slot)
        out_cp[0].wait()
        out_cp[1].wait()

    return k(x)


# speedup vs baseline: 17.4095x; 1.0025x over previous
import functools

import jax
import jax.numpy as jnp
from jax import lax
from jax.experimental import pallas as pl
from jax.experimental.pallas import tpu as pltpu
from jax.experimental.pallas import tpu_sc as plsc

LEN = 128
NW = 32


def kernel(x):
    T, C, H, W = x.shape
    chunks = LEN * C
    per_w = chunks // NW
    a, b = 2 * (T - 1), 2 * (LEN - 1)

    mesh = plsc.VectorSubcoreMesh(core_axis_name="c", subcore_axis_name="s")

    @functools.partial(
        pl.kernel,
        out_type=jax.ShapeDtypeStruct((LEN, C, H, W), x.dtype),
        mesh=mesh,
        scratch_types=[
            pltpu.VMEM((2, H, W), x.dtype),
            pltpu.SemaphoreType.DMA,
            pltpu.SemaphoreType.DMA,
            pltpu.SemaphoreType.DMA,
            pltpu.SemaphoreType.DMA,
        ],
        compiler_params=pltpu.CompilerParams(use_tc_tiling_on_sc=True),
    )
    def k(x_hbm, out_hbm, buf, si0, si1, so0, so1):
        wid = lax.axis_index("s") * 2 + lax.axis_index("c")
        base = wid * per_w
        sin = (si0, si1)
        sout = (so0, so1)

        def start_in(q, slot):
            o = base + q
            frame = o // C
            ch = o % C
            src = (frame * a + (LEN - 1)) // b
            return pltpu.async_copy(x_hbm.at[src, ch], buf.at[slot], sin[slot])

        def start_out(q, slot):
            o = base + q
            return pltpu.async_copy(buf.at[slot], out_hbm.at[o // C, o % C], sout[slot])

        in_cp = [None, None]
        out_cp = [None, None]
        in_cp[0] = start_in(0, 0)
        for q in range(per_w):
            slot = q % 2
            nxt = (q + 1) % 2
            if q + 1 < per_w:
                if q >= 1:
                    out_cp[nxt].wait()
                in_cp[nxt] = start_in(q + 1, nxt)
            in_cp[slot].wait()
            out_cp[slot] = start_out(q, slot)
        out_cp[0].wait()
        out_cp[1].wait()

    return k(x)
